# baseline (speedup 1.0000x reference)
import functools

import jax
import jax.numpy as jnp
from jax import lax
from jax.experimental import pallas as pl
from jax.experimental.pallas import tpu as pltpu
from jax.experimental.pallas import tpu_sc as plsc

NUM_HID = 64
BATCH = 1024
SEQ_LEN = 200

_NC = 2
_NS = 16
_NW = _NC * _NS
_SEQ_PER_W = BATCH // _NW
_CHUNK = 4
_NCHUNK = _SEQ_PER_W // _CHUNK
_ROWS = _CHUNK * SEQ_LEN


def _pos_encoding():
    positions = jnp.arange(SEQ_LEN, dtype=jnp.float32)[:, None]
    depth = NUM_HID / 2
    depths = jnp.arange(depth, dtype=jnp.float32)[None, :] / depth
    angle_rates = 1.0 / (10000.0 ** depths)
    angle_rads = positions * angle_rates
    return jnp.concatenate(
        [jnp.sin(angle_rads), jnp.cos(angle_rads)], axis=-1)


def _sc_body(x_hbm, tab_hbm, pe_hbm, out_hbm,
             idx0, idx1, rows0, rows1, pe_sh,
             sem_g0, sem_g1, sem_s0, sem_s1):
    c = lax.axis_index("c")
    s = lax.axis_index("s")
    wid = s * _NC + c

    @pl.when(s == 0)
    def _stage():
        pltpu.sync_copy(pe_hbm, rows0.at[pl.ds(0, SEQ_LEN)])
        for k in range(_CHUNK):
            pltpu.sync_copy(rows0.at[pl.ds(0, SEQ_LEN)],
                            pe_sh.at[pl.ds(k * SEQ_LEN, SEQ_LEN)])
    plsc.subcore_barrier()

    idxs = (idx0, idx1)
    rows = (rows0, rows1)
    sem_g = (sem_g0, sem_g1)
    sem_s = (sem_s0, sem_s1)
    gather_d = [None, None]
    scatter_d = [None, None]
    seq_w = wid * _SEQ_PER_W

    def scatter_chunk(g, b):
        seq0 = seq_w + g * _CHUNK
        d = None
        for k in range(_CHUNK):
            d = pltpu.async_copy(
                rows[b].at[pl.ds(k * SEQ_LEN, SEQ_LEN)],
                out_hbm.at[seq0 + k], sem_s[b])
        return d

    for g in range(_NCHUNK):
        b = g & 1
        base = (seq_w + g * _CHUNK) * SEQ_LEN
        if scatter_d[b] is not None:
            for _ in range(_CHUNK):
                scatter_d[b].wait()
        pltpu.sync_copy(x_hbm.at[pl.ds(base, _ROWS)], idxs[b])
        pltpu.sync_copy(pe_sh, rows[b])
        gather_d[b] = pltpu.async_copy(
            tab_hbm.at[idxs[b]], rows[b], sem_g[b], add=True)
        if g > 0:
            pb = 1 - b
            gather_d[pb].wait()
            scatter_d[pb] = scatter_chunk(g - 1, pb)

    last = (_NCHUNK - 1) & 1
    gather_d[last].wait()
    scatter_d[last] = scatter_chunk(_NCHUNK - 1, last)
    for _ in range(_CHUNK):
        scatter_d[1 - last].wait()
        scatter_d[last].wait()


@jax.jit
def _run(x_flat, emb_table, pe):
    mesh = plsc.VectorSubcoreMesh(
        core_axis_name="c", subcore_axis_name="s",
        num_cores=_NC, num_subcores=_NS)
    kern = functools.partial(
        pl.kernel,
        out_type=jax.ShapeDtypeStruct((BATCH, SEQ_LEN, NUM_HID), jnp.float32),
        mesh=mesh,
        scratch_types=[
            pltpu.VMEM((_ROWS,), jnp.int32),
            pltpu.VMEM((_ROWS,), jnp.int32),
            pltpu.VMEM((_ROWS, NUM_HID), jnp.float32),
            pltpu.VMEM((_ROWS, NUM_HID), jnp.float32),
            pltpu.VMEM_SHARED((_ROWS, NUM_HID), jnp.float32),
            pltpu.SemaphoreType.DMA,
            pltpu.SemaphoreType.DMA,
            pltpu.SemaphoreType.DMA,
            pltpu.SemaphoreType.DMA,
        ],
        compiler_params=pltpu.CompilerParams(use_tc_tiling_on_sc=False),
    )(_sc_body)
    return kern(x_flat, emb_table, pe)


def kernel(x, emb_table):
    pe = _pos_encoding()
    x_flat = x.reshape(-1).astype(jnp.int32)
    return _run(x_flat, emb_table, pe)
